# baseline (speedup 1.0000x reference)
import functools

import jax
import jax.numpy as jnp
from jax import lax
from jax.experimental import pallas as pl
from jax.experimental.pallas import tpu as pltpu
from jax.experimental.pallas import tpu_sc as plsc

N = 10000
E = 320000
NG = 128
EPS = 1e-15

NC = 2
NS = 16
CHUNK = 80
KS = 5
ISLAB = 25
RING = 4
NP = 10240
ROWS_PER_TILE = NP // NS

_MESH = plsc.VectorSubcoreMesh(
    core_axis_name="c", subcore_axis_name="s", num_cores=NC, num_subcores=NS)


def _sc_agg(table, srcs_flat, dst, width, colsplit):
    ept = E // NS if colsplit else E // (NC * NS)
    nchunk = ept // CHUNK
    nslab = nchunk // ISLAB

    def body(table_h, srcs_h, dst_h, zeros_h, out_h, acc, idx_s, idx_d, rows,
             sm0, sm1, sm2, sm3):
        sems = [sm0, sm1, sm2, sm3]
        c = lax.axis_index("c")
        s = lax.axis_index("s")
        row0 = s * ROWS_PER_TILE

        pltpu.sync_copy(zeros_h, acc.at[pl.ds(row0, ROWS_PER_TILE)])
        plsc.subcore_barrier()

        if colsplit:
            src_base = c * E + s * ept
            dst_base = s * ept
        else:
            w = c * NS + s
            src_base = w * ept
            dst_base = w * ept

        def fire(j):
            return pltpu.async_copy(
                table_h.at[idx_s.at[pl.ds(j * CHUNK, CHUNK)]],
                rows.at[pl.ds((j % RING) * CHUNK, CHUNK)], sems[j % RING])

        def slab(m, _):
            off = m * (ISLAB * CHUNK)
            pltpu.sync_copy(srcs_h.at[pl.ds(src_base + off, ISLAB * CHUNK)], idx_s)
            pltpu.sync_copy(dst_h.at[pl.ds(dst_base + off, ISLAB * CHUNK)], idx_d)
            cps = {}
            for j in range(RING - 1):
                cps[j] = fire(j)
            for j in range(ISLAB):
                if j + RING - 1 < ISLAB:
                    cps[j + RING - 1] = fire(j + RING - 1)
                cps[j].wait()
                pltpu.sync_copy(rows.at[pl.ds((j % RING) * CHUNK, CHUNK)],
                                acc.at[idx_d.at[pl.ds(j * CHUNK, CHUNK)]],
                                add=True)
            return 0
        lax.fori_loop(0, nslab, slab, 0)
        plsc.subcore_barrier()

        out_base = c * NP + row0
        for j in range(ROWS_PER_TILE // 128):
            pltpu.sync_copy(acc.at[pl.ds(row0 + j * 128, 128)],
                            out_h.at[pl.ds(out_base + j * 128, 128)])

    k = pl.kernel(
        body,
        out_type=jax.ShapeDtypeStruct((NC * NP, width), jnp.float32),
        mesh=_MESH,
        compiler_params=pltpu.CompilerParams(use_tc_tiling_on_sc=False),
        scratch_types=[
            pltpu.VMEM_SHARED((NP, width), jnp.float32),
            pltpu.VMEM((ISLAB * CHUNK,), jnp.int32),
            pltpu.VMEM((ISLAB * CHUNK,), jnp.int32),
            pltpu.VMEM((RING * CHUNK, width), jnp.float32),
            pltpu.SemaphoreType.DMA,
            pltpu.SemaphoreType.DMA,
            pltpu.SemaphoreType.DMA,
            pltpu.SemaphoreType.DMA,
        ],
    )
    zeros = jnp.zeros((ROWS_PER_TILE, width), jnp.float32)
    return k(table, srcs_flat, dst, zeros)


def _sc_edge_dot(ts, td, ia_flat, ib_flat, ne):
    ept = ne // (NC * NS)
    nchunk = ept // CHUNK
    ngroup = CHUNK // 16
    wdt = 80
    KD = 4
    SLABE = KD * CHUNK
    nslab = nchunk // KD

    def body(ts_h, td_h, ia_h, ib_h, out_h, idx_a, idx_b, rows_a, rows_b,
             dbuf, sem_a, sem_b):
        c = lax.axis_index("c")
        s = lax.axis_index("s")
        w = c * NS + s
        base = w * ept
        lanes = lax.iota(jnp.int32, 16)

        def load_and_fire(m):
            half = (m % 2) * SLABE
            off = base + m * SLABE
            pltpu.sync_copy(ia_h.at[pl.ds(off, SLABE)],
                            idx_a.at[pl.ds(half, SLABE)])
            pltpu.sync_copy(ib_h.at[pl.ds(off, SLABE)],
                            idx_b.at[pl.ds(half, SLABE)])
            for j in range(KD):
                sl = pl.ds(half + j * CHUNK, CHUNK)
                pltpu.async_copy(ts_h.at[idx_a.at[sl]], rows_a.at[sl], sem_a)
                pltpu.async_copy(td_h.at[idx_b.at[sl]], rows_b.at[sl], sem_b)

        load_and_fire(0)

        def slab(m, _):
            half = (m % 2) * SLABE

            @pl.when(m < nslab - 1)
            def _():
                load_and_fire(m + 1)

            for j in range(KD):
                sl = pl.ds(half + j * CHUNK, CHUNK)
                pltpu.make_async_copy(ts_h.at[idx_a.at[sl]], rows_a.at[sl],
                                      sem_a).wait()
                pltpu.make_async_copy(td_h.at[idx_b.at[sl]], rows_b.at[sl],
                                      sem_b).wait()
                for g in range(ngroup):
                    rows16 = (half + j * CHUNK + g * 16) + lanes
                    accs = [jnp.zeros((16,), jnp.float32) for _ in range(4)]
                    for f in range(66):
                        col = jnp.full((16,), f, jnp.int32)
                        a = plsc.load_gather(rows_a, [rows16, col])
                        b = plsc.load_gather(rows_b, [rows16, col])
                        accs[f % 4] = accs[f % 4] + a * b
                    dbuf[pl.ds(j * CHUNK + g * 16, 16)] = (
                        (accs[0] + accs[1]) + (accs[2] + accs[3]))
            pltpu.sync_copy(dbuf, out_h.at[pl.ds(base + m * SLABE, SLABE)])
            return 0
        lax.fori_loop(0, nslab, slab, 0)

    k = pl.kernel(
        body,
        out_type=jax.ShapeDtypeStruct((ne,), jnp.float32),
        mesh=_MESH,
        compiler_params=pltpu.CompilerParams(use_tc_tiling_on_sc=False,
                                             needs_layout_passes=False),
        scratch_types=[
            pltpu.VMEM((2 * SLABE,), jnp.int32),
            pltpu.VMEM((2 * SLABE,), jnp.int32),
            pltpu.VMEM((2 * SLABE, wdt), jnp.float32),
            pltpu.VMEM((2 * SLABE, wdt), jnp.float32),
            pltpu.VMEM((SLABE,), jnp.float32),
            pltpu.SemaphoreType.DMA,
            pltpu.SemaphoreType.DMA,
        ],
    )
    return k(ts, td, ia_flat, ib_flat)


def kernel(x, edge_index, batch, num_graphs, W0, b0, W1, b1, Wnm, bnm, Wnl, bnl,
           Wcm, bcm, Wcl, bcl, Wd1, bd1, Wd2, bd2):
    src = edge_index[0]
    dst = edge_index[1]

    p = _sc_agg(x, src, dst, 128, colsplit=False)
    agg1 = p[:N] + p[NP:NP + N]
    h1 = jax.nn.relu((x + agg1) @ W0 + b0)

    p = _sc_agg(h1, src, dst, 64, colsplit=False)
    agg2 = p[:N] + p[NP:NP + N]
    h2 = jax.nn.relu((h1 + agg2) @ W1 + b1)

    emb = jnp.concatenate([h1, h2], axis=1)
    node_mu = emb @ Wnm + bnm
    node_logvar = emb @ Wnl + bnl
    class_mu = emb @ Wcm + bcm
    class_logvar = emb @ Wcl + bcl
    node_kl = -0.5 * jnp.sum(1.0 + node_logvar - node_mu ** 2 - jnp.exp(node_logvar))
    node_kl = 1e-07 * node_kl * num_graphs
    inv_var = jnp.exp(-class_logvar)
    T = jax.ops.segment_sum(inv_var, batch, num_segments=NG)
    Smu = jax.ops.segment_sum(class_mu * inv_var, batch, num_segments=NG)
    gvar = 1.0 / (T + 1e-07)
    gmu = gvar * Smu
    glogvar = jnp.log(gvar + EPS)
    class_kl = -0.5 * jnp.sum(1.0 + glogvar - gmu ** 2 - jnp.exp(glogvar))
    class_kl = 1e-07 * class_kl * num_graphs
    kz = jax.random.key(42)
    eps_n = jax.random.normal(jax.random.fold_in(kz, 0), node_mu.shape, node_mu.dtype)
    z_node = node_mu + eps_n * jnp.exp(0.5 * node_logvar)
    eps_g = jax.random.normal(jax.random.fold_in(kz, 1), gmu.shape, gmu.dtype)
    z_class = gmu[batch] + eps_g[batch] * jnp.exp(0.5 * glogvar)[batch]

    tbl = jnp.concatenate([z_node, z_class], axis=0)
    srcs2 = jnp.concatenate([src, src + N])
    p = _sc_agg(tbl, srcs2, dst, 128, colsplit=True)
    agg3 = jnp.concatenate([p[:N], p[NP:NP + N]], axis=1)

    h = jnp.concatenate([z_node, z_class], axis=1)
    h1d = jax.nn.relu((h + agg3) @ Wd1 + bd1)

    M = Wd2 @ Wd2.T
    v = Wd2 @ bd2
    cc = jnp.dot(bd2, bd2)
    P = h1d @ M
    q = h1d @ v + 0.5 * cc
    one = jnp.ones((N, 1), jnp.float32)
    zpad = jnp.zeros((N, 14), jnp.float32)
    ts = jnp.concatenate([P, q[:, None], one, zpad], axis=1)
    td = jnp.concatenate([h1d, one, q[:, None], zpad], axis=1)

    neg = jax.random.randint(jax.random.fold_in(kz, 2), (2, E), 0, N, dtype=jnp.int32)
    ne_pad = 655360
    zpad_i = jnp.zeros((ne_pad - 2 * E,), jnp.int32)
    ia = jnp.concatenate([src, neg[0], zpad_i])
    ib = jnp.concatenate([dst, neg[1], zpad_i])
    d = _sc_edge_dot(ts, td, ia, ib, ne_pad)
    pos = jax.nn.sigmoid(d[:E])
    pos_loss = -jnp.mean(jnp.log(pos + EPS))
    nv = jax.nn.sigmoid(d[E:])
    neg_loss = -jnp.mean(jnp.log(1.0 - nv + EPS))
    recon_err = pos_loss + neg_loss

    return jnp.stack([recon_err, class_kl, node_kl])


# speedup vs baseline: 1.0700x; 1.0700x over previous
import functools

import jax
import jax.numpy as jnp
from jax import lax
from jax.experimental import pallas as pl
from jax.experimental.pallas import tpu as pltpu
from jax.experimental.pallas import tpu_sc as plsc

N = 10000
E = 320000
NG = 128
EPS = 1e-15

NC = 2
NS = 16
CHUNK = 80
KS = 5
ISLAB = 25
RING = 4
NP = 10240
ROWS_PER_TILE = NP // NS

_MESH = plsc.VectorSubcoreMesh(
    core_axis_name="c", subcore_axis_name="s", num_cores=NC, num_subcores=NS)


def _sc_agg(table, srcs_flat, dst, width, colsplit):
    ept = E // NS if colsplit else E // (NC * NS)
    nchunk = ept // CHUNK
    nslab = nchunk // ISLAB

    def body(table_h, srcs_h, dst_h, zeros_h, out_h, acc, idx_s, idx_d, rows,
             sm0, sm1, sm2, sm3):
        sems = [sm0, sm1, sm2, sm3]
        c = lax.axis_index("c")
        s = lax.axis_index("s")
        row0 = s * ROWS_PER_TILE

        pltpu.sync_copy(zeros_h, acc.at[pl.ds(row0, ROWS_PER_TILE)])
        plsc.subcore_barrier()

        if colsplit:
            src_base = c * E + s * ept
            dst_base = s * ept
        else:
            w = c * NS + s
            src_base = w * ept
            dst_base = w * ept

        def fire(j):
            return pltpu.async_copy(
                table_h.at[idx_s.at[pl.ds(j * CHUNK, CHUNK)]],
                rows.at[pl.ds((j % RING) * CHUNK, CHUNK)], sems[j % RING])

        def slab(m, _):
            off = m * (ISLAB * CHUNK)
            pltpu.sync_copy(srcs_h.at[pl.ds(src_base + off, ISLAB * CHUNK)], idx_s)
            pltpu.sync_copy(dst_h.at[pl.ds(dst_base + off, ISLAB * CHUNK)], idx_d)
            cps = {}
            for j in range(RING - 1):
                cps[j] = fire(j)
            for j in range(ISLAB):
                if j + RING - 1 < ISLAB:
                    cps[j + RING - 1] = fire(j + RING - 1)
                cps[j].wait()
                pltpu.sync_copy(rows.at[pl.ds((j % RING) * CHUNK, CHUNK)],
                                acc.at[idx_d.at[pl.ds(j * CHUNK, CHUNK)]],
                                add=True)
            return 0
        lax.fori_loop(0, nslab, slab, 0)
        plsc.subcore_barrier()

        out_base = c * NP + row0
        for j in range(ROWS_PER_TILE // 128):
            pltpu.sync_copy(acc.at[pl.ds(row0 + j * 128, 128)],
                            out_h.at[pl.ds(out_base + j * 128, 128)])

    k = pl.kernel(
        body,
        out_type=jax.ShapeDtypeStruct((NC * NP, width), jnp.float32),
        mesh=_MESH,
        compiler_params=pltpu.CompilerParams(use_tc_tiling_on_sc=False),
        scratch_types=[
            pltpu.VMEM_SHARED((NP, width), jnp.float32),
            pltpu.VMEM((ISLAB * CHUNK,), jnp.int32),
            pltpu.VMEM((ISLAB * CHUNK,), jnp.int32),
            pltpu.VMEM((RING * CHUNK, width), jnp.float32),
            pltpu.SemaphoreType.DMA,
            pltpu.SemaphoreType.DMA,
            pltpu.SemaphoreType.DMA,
            pltpu.SemaphoreType.DMA,
        ],
    )
    zeros = jnp.zeros((ROWS_PER_TILE, width), jnp.float32)
    return k(table, srcs_flat, dst, zeros)


def _sc_edge_dot(ts, td, ia_flat, ib_flat, ne):
    ept = ne // (NC * NS)
    nchunk = ept // CHUNK
    ngroup = CHUNK // 16
    wdt = 80
    KD = 4
    SLABE = KD * CHUNK
    nslab = nchunk // KD

    def body(ts_h, td_h, ia_h, ib_h, out_h, idx_a, idx_b, rows_a, rows_b,
             dbuf, sem_a, sem_b):
        c = lax.axis_index("c")
        s = lax.axis_index("s")
        w = c * NS + s
        base = w * ept
        lanes = lax.iota(jnp.int32, 16)

        def load_and_fire(m):
            half = (m % 2) * SLABE
            off = base + m * SLABE
            pltpu.sync_copy(ia_h.at[pl.ds(off, SLABE)],
                            idx_a.at[pl.ds(half, SLABE)])
            pltpu.sync_copy(ib_h.at[pl.ds(off, SLABE)],
                            idx_b.at[pl.ds(half, SLABE)])
            for j in range(KD):
                sl = pl.ds(half + j * CHUNK, CHUNK)
                pltpu.async_copy(ts_h.at[idx_a.at[sl]], rows_a.at[sl], sem_a)
                pltpu.async_copy(td_h.at[idx_b.at[sl]], rows_b.at[sl], sem_b)

        load_and_fire(0)

        def slab(m, _):
            half = (m % 2) * SLABE

            @pl.when(m < nslab - 1)
            def _():
                load_and_fire(m + 1)

            for j in range(KD):
                sl = pl.ds(half + j * CHUNK, CHUNK)
                pltpu.make_async_copy(ts_h.at[idx_a.at[sl]], rows_a.at[sl],
                                      sem_a).wait()
                pltpu.make_async_copy(td_h.at[idx_b.at[sl]], rows_b.at[sl],
                                      sem_b).wait()
                for g in range(ngroup):
                    rows16 = (half + j * CHUNK + g * 16) + lanes
                    accs = [jnp.zeros((16,), jnp.float32) for _ in range(4)]
                    for f in range(66):
                        col = jnp.full((16,), f, jnp.int32)
                        a = plsc.load_gather(rows_a, [rows16, col])
                        b = plsc.load_gather(rows_b, [rows16, col])
                        accs[f % 4] = accs[f % 4] + a * b
                    dbuf[pl.ds(j * CHUNK + g * 16, 16)] = (
                        (accs[0] + accs[1]) + (accs[2] + accs[3]))
            pltpu.sync_copy(dbuf, out_h.at[pl.ds(base + m * SLABE, SLABE)])
            return 0
        lax.fori_loop(0, nslab, slab, 0)

    k = pl.kernel(
        body,
        out_type=jax.ShapeDtypeStruct((ne,), jnp.float32),
        mesh=_MESH,
        compiler_params=pltpu.CompilerParams(use_tc_tiling_on_sc=False,
                                             needs_layout_passes=False),
        scratch_types=[
            pltpu.VMEM((2 * SLABE,), jnp.int32),
            pltpu.VMEM((2 * SLABE,), jnp.int32),
            pltpu.VMEM((2 * SLABE, wdt), jnp.float32),
            pltpu.VMEM((2 * SLABE, wdt), jnp.float32),
            pltpu.VMEM((SLABE,), jnp.float32),
            pltpu.SemaphoreType.DMA,
            pltpu.SemaphoreType.DMA,
        ],
    )
    return k(ts, td, ia_flat, ib_flat)


NB = 2000
NBLK = N // NB
_HI = jax.lax.Precision.HIGHEST


def _tc_layer(x, p0, p1, W, b):
    dw = W.shape[1]
    din = x.shape[1]

    def body(x_r, p0_r, p1_r, w_r, b_r, o_r):
        acc = x_r[...] + p0_r[...] + p1_r[...]
        o_r[...] = jnp.maximum(
            jnp.dot(acc, w_r[...], precision=_HI) + b_r[...], 0.0)

    return pl.pallas_call(
        body,
        grid=(NBLK,),
        in_specs=[
            pl.BlockSpec((NB, din), lambda i: (i, 0)),
            pl.BlockSpec((NB, din), lambda i: (i, 0)),
            pl.BlockSpec((NB, din), lambda i: (i, 0)),
            pl.BlockSpec((din, dw), lambda i: (0, 0)),
            pl.BlockSpec((1, dw), lambda i: (0, 0)),
        ],
        out_specs=pl.BlockSpec((NB, dw), lambda i: (i, 0)),
        out_shape=jax.ShapeDtypeStruct((N, dw), jnp.float32),
    )(x, p0, p1, W, b.reshape(1, dw))


def _tc_heads(h1, p20, p21, W1, b1, Wnm, bnm, Wnl, bnl, Wcm, bcm, Wcl, bcl,
              eps_n, batch2):
    def body(h1_r, p20_r, p21_r, w1_r, b1_r, wnm_r, bnm_r, wnl_r, bnl_r,
             wcm_r, bcm_r, wcl_r, bcl_r, epsn_r, bt_r,
             zn_r, T_r, Smu_r, nkl_r):
        i = pl.program_id(0)
        h1b = h1_r[...]
        h2 = jnp.maximum(
            jnp.dot(h1b + p20_r[...] + p21_r[...], w1_r[...], precision=_HI)
            + b1_r[...], 0.0)
        emb = jnp.concatenate([h1b, h2], axis=1)
        nm = jnp.dot(emb, wnm_r[...], precision=_HI) + bnm_r[...]
        nl = jnp.dot(emb, wnl_r[...], precision=_HI) + bnl_r[...]
        zn_r[...] = nm + epsn_r[...] * jnp.exp(0.5 * nl)
        nkl_c = jnp.sum(1.0 + nl - nm * nm - jnp.exp(nl), axis=0,
                        keepdims=True)
        cm = jnp.dot(emb, wcm_r[...], precision=_HI) + bcm_r[...]
        cl = jnp.dot(emb, wcl_r[...], precision=_HI) + bcl_r[...]
        iv = jnp.exp(-cl)
        bt = jnp.broadcast_to(bt_r[0], (NG, NB))
        gid = jax.lax.broadcasted_iota(jnp.int32, (NG, NB), 0)
        oht = (bt == gid).astype(jnp.float32)
        T_c = jnp.dot(oht, iv, precision=_HI)
        Smu_c = jnp.dot(oht, cm * iv, precision=_HI)

        @pl.when(i == 0)
        def _():
            nkl_r[...] = nkl_c
            T_r[...] = T_c
            Smu_r[...] = Smu_c

        @pl.when(i > 0)
        def _():
            nkl_r[...] += nkl_c
            T_r[...] += T_c
            Smu_r[...] += Smu_c

    full64 = pl.BlockSpec((64, 128), lambda i: (0, 0))
    row128 = pl.BlockSpec((1, 128), lambda i: (0, 0))
    full128 = pl.BlockSpec((128, 128), lambda i: (0, 0))
    return pl.pallas_call(
        body,
        grid=(NBLK,),
        in_specs=[
            pl.BlockSpec((NB, 64), lambda i: (i, 0)),
            pl.BlockSpec((NB, 64), lambda i: (i, 0)),
            pl.BlockSpec((NB, 64), lambda i: (i, 0)),
            pl.BlockSpec((64, 64), lambda i: (0, 0)),
            pl.BlockSpec((1, 64), lambda i: (0, 0)),
            full128, row128, full128, row128, full128, row128, full128, row128,
            pl.BlockSpec((NB, 128), lambda i: (i, 0)),
            pl.BlockSpec((1, 1, NB), lambda i: (i, 0, 0)),
        ],
        out_specs=[
            pl.BlockSpec((NB, 128), lambda i: (i, 0)),
            pl.BlockSpec((NG, 128), lambda i: (0, 0)),
            pl.BlockSpec((NG, 128), lambda i: (0, 0)),
            pl.BlockSpec((1, 128), lambda i: (0, 0)),
        ],
        out_shape=[
            jax.ShapeDtypeStruct((N, 128), jnp.float32),
            jax.ShapeDtypeStruct((NG, 128), jnp.float32),
            jax.ShapeDtypeStruct((NG, 128), jnp.float32),
            jax.ShapeDtypeStruct((1, 128), jnp.float32),
        ],
    )(h1, p20, p21, W1, b1.reshape(1, 64), Wnm, bnm.reshape(1, 128),
      Wnl, bnl.reshape(1, 128), Wcm, bcm.reshape(1, 128),
      Wcl, bcl.reshape(1, 128), eps_n, batch2)


def _tc_zclass(T, Smu, eps_g, batch2):
    def body(T_r, Smu_r, epsg_r, bt_r, zc_r, ckl_r):
        gvar = 1.0 / (T_r[...] + 1e-07)
        gmu = gvar * Smu_r[...]
        glog = jnp.log(gvar + EPS)
        ZG = gmu + epsg_r[...] * jnp.exp(0.5 * glog)
        ckl_r[...] = jnp.sum(1.0 + glog - gmu * gmu - jnp.exp(glog), axis=0,
                             keepdims=True)
        bt = jnp.broadcast_to(bt_r[0], (NG, NB))
        gid = jax.lax.broadcasted_iota(jnp.int32, (NG, NB), 0)
        oht = (bt == gid).astype(jnp.float32)
        zc_r[...] = jax.lax.dot_general(
            oht, ZG, (((0,), (0,)), ((), ())), precision=_HI)

    ngspec = pl.BlockSpec((NG, 128), lambda i: (0, 0))
    return pl.pallas_call(
        body,
        grid=(NBLK,),
        in_specs=[ngspec, ngspec, ngspec,
                  pl.BlockSpec((1, 1, NB), lambda i: (i, 0, 0))],
        out_specs=[
            pl.BlockSpec((NB, 128), lambda i: (i, 0)),
            pl.BlockSpec((1, 128), lambda i: (0, 0)),
        ],
        out_shape=[
            jax.ShapeDtypeStruct((N, 128), jnp.float32),
            jax.ShapeDtypeStruct((1, 128), jnp.float32),
        ],
    )(T, Smu, eps_g, batch2)


def _tc_decoder(zn, zc, p30, p31, Wd1a, Wd1b, bd1, Wd2, Wd2T, bd2):
    def body(zn_r, zc_r, p30_r, p31_r, wa_r, wb_r, b_r, w2_r, w2t_r, b2_r,
             ts_r, td_r):
        h1d = jnp.maximum(
            jnp.dot(zn_r[...] + p30_r[...], wa_r[...], precision=_HI)
            + jnp.dot(zc_r[...] + p31_r[...], wb_r[...], precision=_HI)
            + b_r[...], 0.0)
        M = jnp.dot(w2_r[...], w2t_r[...], precision=_HI)
        vcol = jnp.sum(w2_r[...] * b2_r[...], axis=1, keepdims=True)
        cc = jnp.sum(b2_r[...] * b2_r[...])
        P = jnp.dot(h1d, M, precision=_HI)
        q = jnp.dot(h1d, vcol, precision=_HI) + 0.5 * cc
        one = jnp.ones((NB, 1), jnp.float32)
        zpad = jnp.zeros((NB, 14), jnp.float32)
        ts_r[...] = jnp.concatenate([P, q, one, zpad], axis=1)
        td_r[...] = jnp.concatenate([h1d, one, q, zpad], axis=1)

    return pl.pallas_call(
        body,
        grid=(NBLK,),
        in_specs=[
            pl.BlockSpec((NB, 128), lambda i: (i, 0)),
            pl.BlockSpec((NB, 128), lambda i: (i, 0)),
            pl.BlockSpec((NB, 128), lambda i: (i, 0)),
            pl.BlockSpec((NB, 128), lambda i: (i, 0)),
            pl.BlockSpec((128, 64), lambda i: (0, 0)),
            pl.BlockSpec((128, 64), lambda i: (0, 0)),
            pl.BlockSpec((1, 64), lambda i: (0, 0)),
            pl.BlockSpec((64, 128), lambda i: (0, 0)),
            pl.BlockSpec((128, 64), lambda i: (0, 0)),
            pl.BlockSpec((1, 128), lambda i: (0, 0)),
        ],
        out_specs=[
            pl.BlockSpec((NB, 80), lambda i: (i, 0)),
            pl.BlockSpec((NB, 80), lambda i: (i, 0)),
        ],
        out_shape=[
            jax.ShapeDtypeStruct((N, 80), jnp.float32),
            jax.ShapeDtypeStruct((N, 80), jnp.float32),
        ],
    )(zn, zc, p30, p31, Wd1a, Wd1b, bd1.reshape(1, 64), Wd2, Wd2T,
      bd2.reshape(1, 128))


def _tc_final(d2, nkl_part, ckl_part, num_graphs):
    EB = 32000
    negrid = E // EB

    def body(d_r, nkl_r, ckl_r, sc_r, o_r, accp, accn):
        i = pl.program_id(0)

        @pl.when(i == 0)
        def _():
            accp[...] = jnp.zeros_like(accp)
            accn[...] = jnp.zeros_like(accn)

        blk = d_r[...].reshape(2, EB // 128, 128)
        dp = blk[0]
        dn = blk[1]
        lp = jnp.log(jax.nn.sigmoid(dp) + EPS)
        ln = jnp.log(1.0 - jax.nn.sigmoid(dn) + EPS)
        accp[...] += jnp.sum(lp, axis=0, keepdims=True)
        accn[...] += jnp.sum(ln, axis=0, keepdims=True)

        @pl.when(i == negrid - 1)
        def _():
            pos_loss = -jnp.sum(accp[...]) / E
            neg_loss = -jnp.sum(accn[...]) / E
            recon = pos_loss + neg_loss
            scale = sc_r[0, 0]
            nkl = scale * jnp.sum(nkl_r[...])
            ckl = scale * jnp.sum(ckl_r[...])
            lane = jax.lax.broadcasted_iota(jnp.int32, (1, 128), 1)
            o_r[...] = jnp.where(
                lane == 0, recon,
                jnp.where(lane == 1, ckl, jnp.where(lane == 2, nkl, 0.0)))

    k_call = pl.pallas_call(
        body,
        grid=(negrid,),
        in_specs=[
            pl.BlockSpec((2, EB), lambda i: (0, i)),
            pl.BlockSpec((1, 128), lambda i: (0, 0)),
            pl.BlockSpec((1, 128), lambda i: (0, 0)),
            pl.BlockSpec((1, 1), lambda i: (0, 0), memory_space=pltpu.SMEM),
        ],
        out_specs=pl.BlockSpec((1, 128), lambda i: (0, 0)),
        out_shape=jax.ShapeDtypeStruct((1, 128), jnp.float32),
        scratch_shapes=[pltpu.VMEM((1, 128), jnp.float32),
                        pltpu.VMEM((1, 128), jnp.float32)],
    )
    scale = jnp.full((1, 1), -0.5e-07, jnp.float32) * jnp.float32(num_graphs)
    return k_call(d2, nkl_part, ckl_part, scale)


def kernel(x, edge_index, batch, num_graphs, W0, b0, W1, b1, Wnm, bnm, Wnl, bnl,
           Wcm, bcm, Wcl, bcl, Wd1, bd1, Wd2, bd2):
    src = edge_index[0]
    dst = edge_index[1]
    batch2 = batch.reshape(NBLK, 1, NB)
    kz = jax.random.key(42)
    eps_n = jax.random.normal(jax.random.fold_in(kz, 0), (N, 128), jnp.float32)
    eps_g = jax.random.normal(jax.random.fold_in(kz, 1), (NG, 128), jnp.float32)
    neg = jax.random.randint(jax.random.fold_in(kz, 2), (2, E), 0, N,
                             dtype=jnp.int32)

    p = _sc_agg(x, src, dst, 128, colsplit=False)
    h1 = _tc_layer(x, p[:N], p[NP:NP + N], W0, b0)

    p = _sc_agg(h1, src, dst, 64, colsplit=False)
    z_node, T, Smu, nkl_part = _tc_heads(
        h1, p[:N], p[NP:NP + N], W1, b1, Wnm, bnm, Wnl, bnl, Wcm, bcm,
        Wcl, bcl, eps_n, batch2)

    z_class, ckl_part = _tc_zclass(T, Smu, eps_g, batch2)

    tbl = jnp.concatenate([z_node, z_class], axis=0)
    srcs2 = jnp.concatenate([src, src + N])
    p = _sc_agg(tbl, srcs2, dst, 128, colsplit=True)

    ts, td = _tc_decoder(z_node, z_class, p[:N], p[NP:NP + N],
                         Wd1[:128], Wd1[128:], bd1, Wd2, Wd2.T, bd2)

    ne_pad = 655360
    zpad_i = jnp.zeros((ne_pad - 2 * E,), jnp.int32)
    ia = jnp.concatenate([src, neg[0], zpad_i])
    ib = jnp.concatenate([dst, neg[1], zpad_i])
    d = _sc_edge_dot(ts, td, ia, ib, ne_pad)

    out_row = _tc_final(d[:2 * E].reshape(2, E), nkl_part, ckl_part,
                        num_graphs)
    return out_row[0, :3]


# speedup vs baseline: 1.1433x; 1.0685x over previous
import functools

import jax
import jax.numpy as jnp
from jax import lax
from jax.experimental import pallas as pl
from jax.experimental.pallas import tpu as pltpu
from jax.experimental.pallas import tpu_sc as plsc

N = 10000
E = 320000
NG = 128
EPS = 1e-15

NC = 2
NS = 16
CHUNK = 80
KS = 5
ISLAB = 25
RING = 4
NP = 10240
ROWS_PER_TILE = NP // NS

_MESH = plsc.VectorSubcoreMesh(
    core_axis_name="c", subcore_axis_name="s", num_cores=NC, num_subcores=NS)


def _sc_agg(table, srcs_flat, dst, width, colsplit):
    ept = E // NS if colsplit else E // (NC * NS)
    nchunk = ept // CHUNK
    nslab = nchunk // ISLAB

    def body(table_h, srcs_h, dst_h, zeros_h, out_h, acc, idx_s, idx_d, rows,
             sm0, sm1, sm2, sm3):
        sems = [sm0, sm1, sm2, sm3]
        c = lax.axis_index("c")
        s = lax.axis_index("s")
        row0 = s * ROWS_PER_TILE

        pltpu.sync_copy(zeros_h, acc.at[pl.ds(row0, ROWS_PER_TILE)])
        plsc.subcore_barrier()

        if colsplit:
            src_base = c * E + s * ept
            dst_base = s * ept
        else:
            w = c * NS + s
            src_base = w * ept
            dst_base = w * ept

        def fire(j):
            return pltpu.async_copy(
                table_h.at[idx_s.at[pl.ds(j * CHUNK, CHUNK)]],
                rows.at[pl.ds((j % RING) * CHUNK, CHUNK)], sems[j % RING])

        def slab(m, _):
            off = m * (ISLAB * CHUNK)
            pltpu.sync_copy(srcs_h.at[pl.ds(src_base + off, ISLAB * CHUNK)], idx_s)
            pltpu.sync_copy(dst_h.at[pl.ds(dst_base + off, ISLAB * CHUNK)], idx_d)
            cps = {}
            for j in range(RING - 1):
                cps[j] = fire(j)
            for j in range(ISLAB):
                if j + RING - 1 < ISLAB:
                    cps[j + RING - 1] = fire(j + RING - 1)
                cps[j].wait()
                pltpu.sync_copy(rows.at[pl.ds((j % RING) * CHUNK, CHUNK)],
                                acc.at[idx_d.at[pl.ds(j * CHUNK, CHUNK)]],
                                add=True)
            return 0
        lax.fori_loop(0, nslab, slab, 0)
        plsc.subcore_barrier()

        out_base = c * NP + row0
        for j in range(ROWS_PER_TILE // 128):
            pltpu.sync_copy(acc.at[pl.ds(row0 + j * 128, 128)],
                            out_h.at[pl.ds(out_base + j * 128, 128)])

    k = pl.kernel(
        body,
        out_type=jax.ShapeDtypeStruct((NC * NP, width), jnp.float32),
        mesh=_MESH,
        compiler_params=pltpu.CompilerParams(use_tc_tiling_on_sc=False),
        scratch_types=[
            pltpu.VMEM_SHARED((NP, width), jnp.float32),
            pltpu.VMEM((ISLAB * CHUNK,), jnp.int32),
            pltpu.VMEM((ISLAB * CHUNK,), jnp.int32),
            pltpu.VMEM((RING * CHUNK, width), jnp.float32),
            pltpu.SemaphoreType.DMA,
            pltpu.SemaphoreType.DMA,
            pltpu.SemaphoreType.DMA,
            pltpu.SemaphoreType.DMA,
        ],
    )
    zeros = jnp.zeros((ROWS_PER_TILE, width), jnp.float32)
    return k(table, srcs_flat, dst, zeros)


def _sc_edge_dot(ts, td, ia_flat, ib_flat, ne):
    ept = ne // (NC * NS)
    nchunk = ept // CHUNK
    ngroup = CHUNK // 16
    wdt = 80
    KD = 4
    SLABE = KD * CHUNK
    nslab = nchunk // KD

    def body(ts_h, td_h, ia_h, ib_h, out_h, idx_a, idx_b, rows_a, rows_b,
             dbuf, sem_a, sem_b, sem_ia, sem_ib):
        c = lax.axis_index("c")
        s = lax.axis_index("s")
        w = c * NS + s
        base = w * ept
        lanes = lax.iota(jnp.int32, 16)

        def load_idx(m, sync):
            third = lax.rem(m, 3) * SLABE
            off = base + m * SLABE
            if sync:
                pltpu.sync_copy(ia_h.at[pl.ds(off, SLABE)],
                                idx_a.at[pl.ds(third, SLABE)])
                pltpu.sync_copy(ib_h.at[pl.ds(off, SLABE)],
                                idx_b.at[pl.ds(third, SLABE)])
            else:
                pltpu.async_copy(ia_h.at[pl.ds(off, SLABE)],
                                 idx_a.at[pl.ds(third, SLABE)], sem_ia)
                pltpu.async_copy(ib_h.at[pl.ds(off, SLABE)],
                                 idx_b.at[pl.ds(third, SLABE)], sem_ib)

        def fire(m):
            third = lax.rem(m, 3) * SLABE
            half = lax.rem(m, 2) * SLABE
            for j in range(KD):
                sli = pl.ds(third + j * CHUNK, CHUNK)
                slr = pl.ds(half + j * CHUNK, CHUNK)
                pltpu.async_copy(ts_h.at[idx_a.at[sli]], rows_a.at[slr], sem_a)
                pltpu.async_copy(td_h.at[idx_b.at[sli]], rows_b.at[slr], sem_b)

        load_idx(0, True)
        load_idx(1, False)
        fire(0)

        sl0 = pl.ds(0, SLABE)
        slc = pl.ds(0, CHUNK)

        def slab(m, _):
            @pl.when(m + 2 < nslab)
            def _():
                load_idx(m + 2, False)

            @pl.when(m + 1 < nslab)
            def _():
                pltpu.make_async_copy(ia_h.at[sl0], idx_a.at[sl0],
                                      sem_ia).wait()
                pltpu.make_async_copy(ib_h.at[sl0], idx_b.at[sl0],
                                      sem_ib).wait()
                fire(m + 1)

            half = lax.rem(m, 2) * SLABE
            for j in range(KD):
                pltpu.make_async_copy(ts_h.at[idx_a.at[slc]],
                                      rows_a.at[slc], sem_a).wait()
                pltpu.make_async_copy(td_h.at[idx_b.at[slc]],
                                      rows_b.at[slc], sem_b).wait()
                for g in range(ngroup):
                    rows16 = (half + j * CHUNK + g * 16) + lanes
                    accs = [jnp.zeros((16,), jnp.float32) for _ in range(4)]
                    for f in range(66):
                        col = jnp.full((16,), f, jnp.int32)
                        a = plsc.load_gather(rows_a, [rows16, col])
                        b = plsc.load_gather(rows_b, [rows16, col])
                        accs[f % 4] = accs[f % 4] + a * b
                    dbuf[pl.ds(j * CHUNK + g * 16, 16)] = (
                        (accs[0] + accs[1]) + (accs[2] + accs[3]))
            pltpu.sync_copy(dbuf, out_h.at[pl.ds(base + m * SLABE, SLABE)])
            return 0
        lax.fori_loop(0, nslab, slab, 0)

    k = pl.kernel(
        body,
        out_type=jax.ShapeDtypeStruct((ne,), jnp.float32),
        mesh=_MESH,
        compiler_params=pltpu.CompilerParams(use_tc_tiling_on_sc=False,
                                             needs_layout_passes=False),
        scratch_types=[
            pltpu.VMEM((3 * SLABE,), jnp.int32),
            pltpu.VMEM((3 * SLABE,), jnp.int32),
            pltpu.VMEM((2 * SLABE, wdt), jnp.float32),
            pltpu.VMEM((2 * SLABE, wdt), jnp.float32),
            pltpu.VMEM((SLABE,), jnp.float32),
            pltpu.SemaphoreType.DMA,
            pltpu.SemaphoreType.DMA,
            pltpu.SemaphoreType.DMA,
            pltpu.SemaphoreType.DMA,
        ],
    )
    return k(ts, td, ia_flat, ib_flat)


NB = 2000
NBLK = N // NB
_HI = jax.lax.Precision.HIGHEST


def _tc_layer(x, p0, p1, W, b):
    dw = W.shape[1]
    din = x.shape[1]

    def body(x_r, p0_r, p1_r, w_r, b_r, o_r):
        acc = x_r[...] + p0_r[...] + p1_r[...]
        o_r[...] = jnp.maximum(
            jnp.dot(acc, w_r[...], precision=_HI) + b_r[...], 0.0)

    return pl.pallas_call(
        body,
        grid=(NBLK,),
        in_specs=[
            pl.BlockSpec((NB, din), lambda i: (i, 0)),
            pl.BlockSpec((NB, din), lambda i: (i, 0)),
            pl.BlockSpec((NB, din), lambda i: (i, 0)),
            pl.BlockSpec((din, dw), lambda i: (0, 0)),
            pl.BlockSpec((1, dw), lambda i: (0, 0)),
        ],
        out_specs=pl.BlockSpec((NB, dw), lambda i: (i, 0)),
        out_shape=jax.ShapeDtypeStruct((N, dw), jnp.float32),
    )(x, p0, p1, W, b.reshape(1, dw))


def _tc_heads(h1, p20, p21, W1, b1, Wnm, bnm, Wnl, bnl, Wcm, bcm, Wcl, bcl,
              eps_n, batch2):
    def body(h1_r, p20_r, p21_r, w1_r, b1_r, wnm_r, bnm_r, wnl_r, bnl_r,
             wcm_r, bcm_r, wcl_r, bcl_r, epsn_r, bt_r,
             zn_r, T_r, Smu_r, nkl_r):
        i = pl.program_id(0)
        h1b = h1_r[...]
        h2 = jnp.maximum(
            jnp.dot(h1b + p20_r[...] + p21_r[...], w1_r[...], precision=_HI)
            + b1_r[...], 0.0)
        emb = jnp.concatenate([h1b, h2], axis=1)
        nm = jnp.dot(emb, wnm_r[...], precision=_HI) + bnm_r[...]
        nl = jnp.dot(emb, wnl_r[...], precision=_HI) + bnl_r[...]
        zn_r[...] = nm + epsn_r[...] * jnp.exp(0.5 * nl)
        nkl_c = jnp.sum(1.0 + nl - nm * nm - jnp.exp(nl), axis=0,
                        keepdims=True)
        cm = jnp.dot(emb, wcm_r[...], precision=_HI) + bcm_r[...]
        cl = jnp.dot(emb, wcl_r[...], precision=_HI) + bcl_r[...]
        iv = jnp.exp(-cl)
        bt = jnp.broadcast_to(bt_r[0], (NG, NB))
        gid = jax.lax.broadcasted_iota(jnp.int32, (NG, NB), 0)
        oht = (bt == gid).astype(jnp.float32)
        T_c = jnp.dot(oht, iv, precision=_HI)
        Smu_c = jnp.dot(oht, cm * iv, precision=_HI)

        @pl.when(i == 0)
        def _():
            nkl_r[...] = nkl_c
            T_r[...] = T_c
            Smu_r[...] = Smu_c

        @pl.when(i > 0)
        def _():
            nkl_r[...] += nkl_c
            T_r[...] += T_c
            Smu_r[...] += Smu_c

    full64 = pl.BlockSpec((64, 128), lambda i: (0, 0))
    row128 = pl.BlockSpec((1, 128), lambda i: (0, 0))
    full128 = pl.BlockSpec((128, 128), lambda i: (0, 0))
    return pl.pallas_call(
        body,
        grid=(NBLK,),
        in_specs=[
            pl.BlockSpec((NB, 64), lambda i: (i, 0)),
            pl.BlockSpec((NB, 64), lambda i: (i, 0)),
            pl.BlockSpec((NB, 64), lambda i: (i, 0)),
            pl.BlockSpec((64, 64), lambda i: (0, 0)),
            pl.BlockSpec((1, 64), lambda i: (0, 0)),
            full128, row128, full128, row128, full128, row128, full128, row128,
            pl.BlockSpec((NB, 128), lambda i: (i, 0)),
            pl.BlockSpec((1, 1, NB), lambda i: (i, 0, 0)),
        ],
        out_specs=[
            pl.BlockSpec((NB, 128), lambda i: (i, 0)),
            pl.BlockSpec((NG, 128), lambda i: (0, 0)),
            pl.BlockSpec((NG, 128), lambda i: (0, 0)),
            pl.BlockSpec((1, 128), lambda i: (0, 0)),
        ],
        out_shape=[
            jax.ShapeDtypeStruct((N, 128), jnp.float32),
            jax.ShapeDtypeStruct((NG, 128), jnp.float32),
            jax.ShapeDtypeStruct((NG, 128), jnp.float32),
            jax.ShapeDtypeStruct((1, 128), jnp.float32),
        ],
    )(h1, p20, p21, W1, b1.reshape(1, 64), Wnm, bnm.reshape(1, 128),
      Wnl, bnl.reshape(1, 128), Wcm, bcm.reshape(1, 128),
      Wcl, bcl.reshape(1, 128), eps_n, batch2)


def _tc_zclass(T, Smu, eps_g, batch2):
    def body(T_r, Smu_r, epsg_r, bt_r, zc_r, ckl_r):
        gvar = 1.0 / (T_r[...] + 1e-07)
        gmu = gvar * Smu_r[...]
        glog = jnp.log(gvar + EPS)
        ZG = gmu + epsg_r[...] * jnp.exp(0.5 * glog)
        ckl_r[...] = jnp.sum(1.0 + glog - gmu * gmu - jnp.exp(glog), axis=0,
                             keepdims=True)
        bt = jnp.broadcast_to(bt_r[0], (NG, NB))
        gid = jax.lax.broadcasted_iota(jnp.int32, (NG, NB), 0)
        oht = (bt == gid).astype(jnp.float32)
        zc_r[...] = jax.lax.dot_general(
            oht, ZG, (((0,), (0,)), ((), ())), precision=_HI)

    ngspec = pl.BlockSpec((NG, 128), lambda i: (0, 0))
    return pl.pallas_call(
        body,
        grid=(NBLK,),
        in_specs=[ngspec, ngspec, ngspec,
                  pl.BlockSpec((1, 1, NB), lambda i: (i, 0, 0))],
        out_specs=[
            pl.BlockSpec((NB, 128), lambda i: (i, 0)),
            pl.BlockSpec((1, 128), lambda i: (0, 0)),
        ],
        out_shape=[
            jax.ShapeDtypeStruct((N, 128), jnp.float32),
            jax.ShapeDtypeStruct((1, 128), jnp.float32),
        ],
    )(T, Smu, eps_g, batch2)


def _tc_decoder(zn, zc, p30, p31, Wd1a, Wd1b, bd1, Wd2, Wd2T, bd2):
    def body(zn_r, zc_r, p30_r, p31_r, wa_r, wb_r, b_r, w2_r, w2t_r, b2_r,
             ts_r, td_r):
        h1d = jnp.maximum(
            jnp.dot(zn_r[...] + p30_r[...], wa_r[...], precision=_HI)
            + jnp.dot(zc_r[...] + p31_r[...], wb_r[...], precision=_HI)
            + b_r[...], 0.0)
        M = jnp.dot(w2_r[...], w2t_r[...], precision=_HI)
        vcol = jnp.sum(w2_r[...] * b2_r[...], axis=1, keepdims=True)
        cc = jnp.sum(b2_r[...] * b2_r[...])
        P = jnp.dot(h1d, M, precision=_HI)
        q = jnp.dot(h1d, vcol, precision=_HI) + 0.5 * cc
        one = jnp.ones((NB, 1), jnp.float32)
        zpad = jnp.zeros((NB, 14), jnp.float32)
        ts_r[...] = jnp.concatenate([P, q, one, zpad], axis=1)
        td_r[...] = jnp.concatenate([h1d, one, q, zpad], axis=1)

    return pl.pallas_call(
        body,
        grid=(NBLK,),
        in_specs=[
            pl.BlockSpec((NB, 128), lambda i: (i, 0)),
            pl.BlockSpec((NB, 128), lambda i: (i, 0)),
            pl.BlockSpec((NB, 128), lambda i: (i, 0)),
            pl.BlockSpec((NB, 128), lambda i: (i, 0)),
            pl.BlockSpec((128, 64), lambda i: (0, 0)),
            pl.BlockSpec((128, 64), lambda i: (0, 0)),
            pl.BlockSpec((1, 64), lambda i: (0, 0)),
            pl.BlockSpec((64, 128), lambda i: (0, 0)),
            pl.BlockSpec((128, 64), lambda i: (0, 0)),
            pl.BlockSpec((1, 128), lambda i: (0, 0)),
        ],
        out_specs=[
            pl.BlockSpec((NB, 80), lambda i: (i, 0)),
            pl.BlockSpec((NB, 80), lambda i: (i, 0)),
        ],
        out_shape=[
            jax.ShapeDtypeStruct((N, 80), jnp.float32),
            jax.ShapeDtypeStruct((N, 80), jnp.float32),
        ],
    )(zn, zc, p30, p31, Wd1a, Wd1b, bd1.reshape(1, 64), Wd2, Wd2T,
      bd2.reshape(1, 128))


def _tc_final(d2, nkl_part, ckl_part, num_graphs):
    EB = 32000
    negrid = E // EB

    def body(d_r, nkl_r, ckl_r, sc_r, o_r, accp, accn):
        i = pl.program_id(0)

        @pl.when(i == 0)
        def _():
            accp[...] = jnp.zeros_like(accp)
            accn[...] = jnp.zeros_like(accn)

        blk = d_r[...].reshape(2, EB // 128, 128)
        dp = blk[0]
        dn = blk[1]
        lp = jnp.log(jax.nn.sigmoid(dp) + EPS)
        ln = jnp.log(1.0 - jax.nn.sigmoid(dn) + EPS)
        accp[...] += jnp.sum(lp, axis=0, keepdims=True)
        accn[...] += jnp.sum(ln, axis=0, keepdims=True)

        @pl.when(i == negrid - 1)
        def _():
            pos_loss = -jnp.sum(accp[...]) / E
            neg_loss = -jnp.sum(accn[...]) / E
            recon = pos_loss + neg_loss
            scale = sc_r[0, 0]
            nkl = scale * jnp.sum(nkl_r[...])
            ckl = scale * jnp.sum(ckl_r[...])
            lane = jax.lax.broadcasted_iota(jnp.int32, (1, 128), 1)
            o_r[...] = jnp.where(
                lane == 0, recon,
                jnp.where(lane == 1, ckl, jnp.where(lane == 2, nkl, 0.0)))

    k_call = pl.pallas_call(
        body,
        grid=(negrid,),
        in_specs=[
            pl.BlockSpec((2, EB), lambda i: (0, i)),
            pl.BlockSpec((1, 128), lambda i: (0, 0)),
            pl.BlockSpec((1, 128), lambda i: (0, 0)),
            pl.BlockSpec((1, 1), lambda i: (0, 0), memory_space=pltpu.SMEM),
        ],
        out_specs=pl.BlockSpec((1, 128), lambda i: (0, 0)),
        out_shape=jax.ShapeDtypeStruct((1, 128), jnp.float32),
        scratch_shapes=[pltpu.VMEM((1, 128), jnp.float32),
                        pltpu.VMEM((1, 128), jnp.float32)],
    )
    scale = jnp.full((1, 1), -0.5e-07, jnp.float32) * jnp.float32(num_graphs)
    return k_call(d2, nkl_part, ckl_part, scale)


def kernel(x, edge_index, batch, num_graphs, W0, b0, W1, b1, Wnm, bnm, Wnl, bnl,
           Wcm, bcm, Wcl, bcl, Wd1, bd1, Wd2, bd2):
    src = edge_index[0]
    dst = edge_index[1]
    batch2 = batch.reshape(NBLK, 1, NB)
    kz = jax.random.key(42)
    eps_n = jax.random.normal(jax.random.fold_in(kz, 0), (N, 128), jnp.float32)
    eps_g = jax.random.normal(jax.random.fold_in(kz, 1), (NG, 128), jnp.float32)
    neg = jax.random.randint(jax.random.fold_in(kz, 2), (2, E), 0, N,
                             dtype=jnp.int32)

    p = _sc_agg(x, src, dst, 128, colsplit=False)
    h1 = _tc_layer(x, p[:N], p[NP:NP + N], W0, b0)

    p = _sc_agg(h1, src, dst, 64, colsplit=False)
    z_node, T, Smu, nkl_part = _tc_heads(
        h1, p[:N], p[NP:NP + N], W1, b1, Wnm, bnm, Wnl, bnl, Wcm, bcm,
        Wcl, bcl, eps_n, batch2)

    z_class, ckl_part = _tc_zclass(T, Smu, eps_g, batch2)

    tbl = jnp.concatenate([z_node, z_class], axis=0)
    srcs2 = jnp.concatenate([src, src + N])
    p = _sc_agg(tbl, srcs2, dst, 128, colsplit=True)

    ts, td = _tc_decoder(z_node, z_class, p[:N], p[NP:NP + N],
                         Wd1[:128], Wd1[128:], bd1, Wd2, Wd2.T, bd2)

    ne_pad = 655360
    zpad_i = jnp.zeros((ne_pad - 2 * E,), jnp.int32)
    ia = jnp.concatenate([src, neg[0], zpad_i])
    ib = jnp.concatenate([dst, neg[1], zpad_i])
    d = _sc_edge_dot(ts, td, ia, ib, ne_pad)

    out_row = _tc_final(d[:2 * E].reshape(2, E), nkl_part, ckl_part,
                        num_graphs)
    return out_row[0, :3]


# speedup vs baseline: 1.1441x; 1.0007x over previous
import functools

import jax
import jax.numpy as jnp
from jax import lax
from jax.experimental import pallas as pl
from jax.experimental.pallas import tpu as pltpu
from jax.experimental.pallas import tpu_sc as plsc

N = 10000
E = 320000
NG = 128
EPS = 1e-15

NC = 2
NS = 16
CHUNK = 80
KS = 5
ISLAB = 25
RING = 4
NP = 10240
ROWS_PER_TILE = NP // NS

_MESH = plsc.VectorSubcoreMesh(
    core_axis_name="c", subcore_axis_name="s", num_cores=NC, num_subcores=NS)


def _sc_agg(table, srcs_flat, dst, width, colsplit):
    ept = E // NS if colsplit else E // (NC * NS)
    nchunk = ept // CHUNK
    nslab = nchunk // ISLAB

    def body(table_h, srcs_h, dst_h, zeros_h, out_h, acc, idx_s, idx_d, rows,
             sm0, sm1, sm2, sm3):
        sems = [sm0, sm1, sm2, sm3]
        c = lax.axis_index("c")
        s = lax.axis_index("s")
        row0 = s * ROWS_PER_TILE

        pltpu.sync_copy(zeros_h, acc.at[pl.ds(row0, ROWS_PER_TILE)])
        plsc.subcore_barrier()

        if colsplit:
            src_base = c * E + s * ept
            dst_base = s * ept
        else:
            w = c * NS + s
            src_base = w * ept
            dst_base = w * ept

        def fire(j):
            return pltpu.async_copy(
                table_h.at[idx_s.at[pl.ds(j * CHUNK, CHUNK)]],
                rows.at[pl.ds((j % RING) * CHUNK, CHUNK)], sems[j % RING])

        def slab(m, _):
            off = m * (ISLAB * CHUNK)
            pltpu.sync_copy(srcs_h.at[pl.ds(src_base + off, ISLAB * CHUNK)], idx_s)
            pltpu.sync_copy(dst_h.at[pl.ds(dst_base + off, ISLAB * CHUNK)], idx_d)
            cps = {}
            for j in range(RING - 1):
                cps[j] = fire(j)
            for j in range(ISLAB):
                if j + RING - 1 < ISLAB:
                    cps[j + RING - 1] = fire(j + RING - 1)
                cps[j].wait()
                pltpu.sync_copy(rows.at[pl.ds((j % RING) * CHUNK, CHUNK)],
                                acc.at[idx_d.at[pl.ds(j * CHUNK, CHUNK)]],
                                add=True)
            return 0
        lax.fori_loop(0, nslab, slab, 0)
        plsc.subcore_barrier()

        out_base = c * NP + row0
        for j in range(ROWS_PER_TILE // 128):
            pltpu.sync_copy(acc.at[pl.ds(row0 + j * 128, 128)],
                            out_h.at[pl.ds(out_base + j * 128, 128)])

    k = pl.kernel(
        body,
        out_type=jax.ShapeDtypeStruct((NC * NP, width), jnp.float32),
        mesh=_MESH,
        compiler_params=pltpu.CompilerParams(use_tc_tiling_on_sc=False),
        scratch_types=[
            pltpu.VMEM_SHARED((NP, width), jnp.float32),
            pltpu.VMEM((ISLAB * CHUNK,), jnp.int32),
            pltpu.VMEM((ISLAB * CHUNK,), jnp.int32),
            pltpu.VMEM((RING * CHUNK, width), jnp.float32),
            pltpu.SemaphoreType.DMA,
            pltpu.SemaphoreType.DMA,
            pltpu.SemaphoreType.DMA,
            pltpu.SemaphoreType.DMA,
        ],
    )
    zeros = jnp.zeros((ROWS_PER_TILE, width), jnp.float32)
    return k(table, srcs_flat, dst, zeros)


def _sc_edge_dot(ts, td, ia_flat, ib_flat, ne):
    ept = ne // (NC * NS)
    nchunk = ept // CHUNK
    ngroup = CHUNK // 16
    wdt = 80
    KD = 4
    SLABE = KD * CHUNK
    nslab = nchunk // KD

    def body(ts_h, td_h, ia_h, ib_h, out_h, idx_a, idx_b, rows_a, rows_b,
             dbuf, sem_a, sem_b, sem_ia, sem_ib, sem_d):
        c = lax.axis_index("c")
        s = lax.axis_index("s")
        w = c * NS + s
        base = w * ept
        lanes = lax.iota(jnp.int32, 16)

        def load_idx(m, sync):
            third = lax.rem(m, 3) * SLABE
            off = base + m * SLABE
            if sync:
                pltpu.sync_copy(ia_h.at[pl.ds(off, SLABE)],
                                idx_a.at[pl.ds(third, SLABE)])
                pltpu.sync_copy(ib_h.at[pl.ds(off, SLABE)],
                                idx_b.at[pl.ds(third, SLABE)])
            else:
                pltpu.async_copy(ia_h.at[pl.ds(off, SLABE)],
                                 idx_a.at[pl.ds(third, SLABE)], sem_ia)
                pltpu.async_copy(ib_h.at[pl.ds(off, SLABE)],
                                 idx_b.at[pl.ds(third, SLABE)], sem_ib)

        def fire(m):
            third = lax.rem(m, 3) * SLABE
            half = lax.rem(m, 2) * SLABE
            for j in range(KD):
                sli = pl.ds(third + j * CHUNK, CHUNK)
                slr = pl.ds(half + j * CHUNK, CHUNK)
                pltpu.async_copy(ts_h.at[idx_a.at[sli]], rows_a.at[slr], sem_a)
                pltpu.async_copy(td_h.at[idx_b.at[sli]], rows_b.at[slr], sem_b)

        load_idx(0, True)
        load_idx(1, False)
        fire(0)

        sl0 = pl.ds(0, SLABE)
        sl0d = sl0
        slc = pl.ds(0, CHUNK)

        def slab(m, _):
            @pl.when(m + 2 < nslab)
            def _():
                load_idx(m + 2, False)

            @pl.when(m + 1 < nslab)
            def _():
                pltpu.make_async_copy(ia_h.at[sl0], idx_a.at[sl0],
                                      sem_ia).wait()
                pltpu.make_async_copy(ib_h.at[sl0], idx_b.at[sl0],
                                      sem_ib).wait()
                fire(m + 1)

            half = lax.rem(m, 2) * SLABE

            @pl.when(m >= 2)
            def _():
                pltpu.make_async_copy(dbuf.at[sl0], out_h.at[sl0d],
                                      sem_d).wait()

            for j in range(KD):
                pltpu.make_async_copy(ts_h.at[idx_a.at[slc]],
                                      rows_a.at[slc], sem_a).wait()
                pltpu.make_async_copy(td_h.at[idx_b.at[slc]],
                                      rows_b.at[slc], sem_b).wait()
                for g in range(ngroup):
                    rows16 = (half + j * CHUNK + g * 16) + lanes
                    accs = [jnp.zeros((16,), jnp.float32) for _ in range(4)]
                    for f in range(66):
                        col = jnp.full((16,), f, jnp.int32)
                        a = plsc.load_gather(rows_a, [rows16, col])
                        b = plsc.load_gather(rows_b, [rows16, col])
                        accs[f % 4] = accs[f % 4] + a * b
                    dbuf[pl.ds(half + j * CHUNK + g * 16, 16)] = (
                        (accs[0] + accs[1]) + (accs[2] + accs[3]))
            pltpu.async_copy(dbuf.at[pl.ds(half, SLABE)],
                             out_h.at[pl.ds(base + m * SLABE, SLABE)], sem_d)
            return 0
        lax.fori_loop(0, nslab, slab, 0)
        pltpu.make_async_copy(dbuf.at[sl0], out_h.at[sl0d], sem_d).wait()
        pltpu.make_async_copy(dbuf.at[sl0], out_h.at[sl0d], sem_d).wait()

    k = pl.kernel(
        body,
        out_type=jax.ShapeDtypeStruct((ne,), jnp.float32),
        mesh=_MESH,
        compiler_params=pltpu.CompilerParams(use_tc_tiling_on_sc=False,
                                             needs_layout_passes=False),
        scratch_types=[
            pltpu.VMEM((3 * SLABE,), jnp.int32),
            pltpu.VMEM((3 * SLABE,), jnp.int32),
            pltpu.VMEM((2 * SLABE, wdt), jnp.float32),
            pltpu.VMEM((2 * SLABE, wdt), jnp.float32),
            pltpu.VMEM((2 * SLABE,), jnp.float32),
            pltpu.SemaphoreType.DMA,
            pltpu.SemaphoreType.DMA,
            pltpu.SemaphoreType.DMA,
            pltpu.SemaphoreType.DMA,
            pltpu.SemaphoreType.DMA,
        ],
    )
    return k(ts, td, ia_flat, ib_flat)


NB = 2000
NBLK = N // NB
_HI = jax.lax.Precision.HIGHEST


def _tc_layer(x, p0, p1, W, b):
    dw = W.shape[1]
    din = x.shape[1]

    def body(x_r, p0_r, p1_r, w_r, b_r, o_r):
        acc = x_r[...] + p0_r[...] + p1_r[...]
        o_r[...] = jnp.maximum(
            jnp.dot(acc, w_r[...], precision=_HI) + b_r[...], 0.0)

    return pl.pallas_call(
        body,
        grid=(NBLK,),
        in_specs=[
            pl.BlockSpec((NB, din), lambda i: (i, 0)),
            pl.BlockSpec((NB, din), lambda i: (i, 0)),
            pl.BlockSpec((NB, din), lambda i: (i, 0)),
            pl.BlockSpec((din, dw), lambda i: (0, 0)),
            pl.BlockSpec((1, dw), lambda i: (0, 0)),
        ],
        out_specs=pl.BlockSpec((NB, dw), lambda i: (i, 0)),
        out_shape=jax.ShapeDtypeStruct((N, dw), jnp.float32),
    )(x, p0, p1, W, b.reshape(1, dw))


def _tc_heads(h1, p20, p21, W1, b1, Wnm, bnm, Wnl, bnl, Wcm, bcm, Wcl, bcl,
              eps_n, batch2):
    def body(h1_r, p20_r, p21_r, w1_r, b1_r, wnm_r, bnm_r, wnl_r, bnl_r,
             wcm_r, bcm_r, wcl_r, bcl_r, epsn_r, bt_r,
             zn_r, T_r, Smu_r, nkl_r):
        i = pl.program_id(0)
        h1b = h1_r[...]
        h2 = jnp.maximum(
            jnp.dot(h1b + p20_r[...] + p21_r[...], w1_r[...], precision=_HI)
            + b1_r[...], 0.0)
        emb = jnp.concatenate([h1b, h2], axis=1)
        nm = jnp.dot(emb, wnm_r[...], precision=_HI) + bnm_r[...]
        nl = jnp.dot(emb, wnl_r[...], precision=_HI) + bnl_r[...]
        zn_r[...] = nm + epsn_r[...] * jnp.exp(0.5 * nl)
        nkl_c = jnp.sum(1.0 + nl - nm * nm - jnp.exp(nl), axis=0,
                        keepdims=True)
        cm = jnp.dot(emb, wcm_r[...], precision=_HI) + bcm_r[...]
        cl = jnp.dot(emb, wcl_r[...], precision=_HI) + bcl_r[...]
        iv = jnp.exp(-cl)
        bt = jnp.broadcast_to(bt_r[0], (NG, NB))
        gid = jax.lax.broadcasted_iota(jnp.int32, (NG, NB), 0)
        oht = (bt == gid).astype(jnp.float32)
        T_c = jnp.dot(oht, iv, precision=_HI)
        Smu_c = jnp.dot(oht, cm * iv, precision=_HI)

        @pl.when(i == 0)
        def _():
            nkl_r[...] = nkl_c
            T_r[...] = T_c
            Smu_r[...] = Smu_c

        @pl.when(i > 0)
        def _():
            nkl_r[...] += nkl_c
            T_r[...] += T_c
            Smu_r[...] += Smu_c

    full64 = pl.BlockSpec((64, 128), lambda i: (0, 0))
    row128 = pl.BlockSpec((1, 128), lambda i: (0, 0))
    full128 = pl.BlockSpec((128, 128), lambda i: (0, 0))
    return pl.pallas_call(
        body,
        grid=(NBLK,),
        in_specs=[
            pl.BlockSpec((NB, 64), lambda i: (i, 0)),
            pl.BlockSpec((NB, 64), lambda i: (i, 0)),
            pl.BlockSpec((NB, 64), lambda i: (i, 0)),
            pl.BlockSpec((64, 64), lambda i: (0, 0)),
            pl.BlockSpec((1, 64), lambda i: (0, 0)),
            full128, row128, full128, row128, full128, row128, full128, row128,
            pl.BlockSpec((NB, 128), lambda i: (i, 0)),
            pl.BlockSpec((1, 1, NB), lambda i: (i, 0, 0)),
        ],
        out_specs=[
            pl.BlockSpec((NB, 128), lambda i: (i, 0)),
            pl.BlockSpec((NG, 128), lambda i: (0, 0)),
            pl.BlockSpec((NG, 128), lambda i: (0, 0)),
            pl.BlockSpec((1, 128), lambda i: (0, 0)),
        ],
        out_shape=[
            jax.ShapeDtypeStruct((N, 128), jnp.float32),
            jax.ShapeDtypeStruct((NG, 128), jnp.float32),
            jax.ShapeDtypeStruct((NG, 128), jnp.float32),
            jax.ShapeDtypeStruct((1, 128), jnp.float32),
        ],
    )(h1, p20, p21, W1, b1.reshape(1, 64), Wnm, bnm.reshape(1, 128),
      Wnl, bnl.reshape(1, 128), Wcm, bcm.reshape(1, 128),
      Wcl, bcl.reshape(1, 128), eps_n, batch2)


def _tc_zclass(T, Smu, eps_g, batch2):
    def body(T_r, Smu_r, epsg_r, bt_r, zc_r, ckl_r):
        gvar = 1.0 / (T_r[...] + 1e-07)
        gmu = gvar * Smu_r[...]
        glog = jnp.log(gvar + EPS)
        ZG = gmu + epsg_r[...] * jnp.exp(0.5 * glog)
        ckl_r[...] = jnp.sum(1.0 + glog - gmu * gmu - jnp.exp(glog), axis=0,
                             keepdims=True)
        bt = jnp.broadcast_to(bt_r[0], (NG, NB))
        gid = jax.lax.broadcasted_iota(jnp.int32, (NG, NB), 0)
        oht = (bt == gid).astype(jnp.float32)
        zc_r[...] = jax.lax.dot_general(
            oht, ZG, (((0,), (0,)), ((), ())), precision=_HI)

    ngspec = pl.BlockSpec((NG, 128), lambda i: (0, 0))
    return pl.pallas_call(
        body,
        grid=(NBLK,),
        in_specs=[ngspec, ngspec, ngspec,
                  pl.BlockSpec((1, 1, NB), lambda i: (i, 0, 0))],
        out_specs=[
            pl.BlockSpec((NB, 128), lambda i: (i, 0)),
            pl.BlockSpec((1, 128), lambda i: (0, 0)),
        ],
        out_shape=[
            jax.ShapeDtypeStruct((N, 128), jnp.float32),
            jax.ShapeDtypeStruct((1, 128), jnp.float32),
        ],
    )(T, Smu, eps_g, batch2)


def _tc_decoder(zn, zc, p30, p31, Wd1a, Wd1b, bd1, Wd2, Wd2T, bd2):
    def body(zn_r, zc_r, p30_r, p31_r, wa_r, wb_r, b_r, w2_r, w2t_r, b2_r,
             ts_r, td_r):
        h1d = jnp.maximum(
            jnp.dot(zn_r[...] + p30_r[...], wa_r[...], precision=_HI)
            + jnp.dot(zc_r[...] + p31_r[...], wb_r[...], precision=_HI)
            + b_r[...], 0.0)
        M = jnp.dot(w2_r[...], w2t_r[...], precision=_HI)
        vcol = jnp.sum(w2_r[...] * b2_r[...], axis=1, keepdims=True)
        cc = jnp.sum(b2_r[...] * b2_r[...])
        P = jnp.dot(h1d, M, precision=_HI)
        q = jnp.dot(h1d, vcol, precision=_HI) + 0.5 * cc
        one = jnp.ones((NB, 1), jnp.float32)
        zpad = jnp.zeros((NB, 14), jnp.float32)
        ts_r[...] = jnp.concatenate([P, q, one, zpad], axis=1)
        td_r[...] = jnp.concatenate([h1d, one, q, zpad], axis=1)

    return pl.pallas_call(
        body,
        grid=(NBLK,),
        in_specs=[
            pl.BlockSpec((NB, 128), lambda i: (i, 0)),
            pl.BlockSpec((NB, 128), lambda i: (i, 0)),
            pl.BlockSpec((NB, 128), lambda i: (i, 0)),
            pl.BlockSpec((NB, 128), lambda i: (i, 0)),
            pl.BlockSpec((128, 64), lambda i: (0, 0)),
            pl.BlockSpec((128, 64), lambda i: (0, 0)),
            pl.BlockSpec((1, 64), lambda i: (0, 0)),
            pl.BlockSpec((64, 128), lambda i: (0, 0)),
            pl.BlockSpec((128, 64), lambda i: (0, 0)),
            pl.BlockSpec((1, 128), lambda i: (0, 0)),
        ],
        out_specs=[
            pl.BlockSpec((NB, 80), lambda i: (i, 0)),
            pl.BlockSpec((NB, 80), lambda i: (i, 0)),
        ],
        out_shape=[
            jax.ShapeDtypeStruct((N, 80), jnp.float32),
            jax.ShapeDtypeStruct((N, 80), jnp.float32),
        ],
    )(zn, zc, p30, p31, Wd1a, Wd1b, bd1.reshape(1, 64), Wd2, Wd2T,
      bd2.reshape(1, 128))


def _tc_final(d2, nkl_part, ckl_part, num_graphs):
    EB = 32000
    negrid = E // EB

    def body(d_r, nkl_r, ckl_r, sc_r, o_r, accp, accn):
        i = pl.program_id(0)

        @pl.when(i == 0)
        def _():
            accp[...] = jnp.zeros_like(accp)
            accn[...] = jnp.zeros_like(accn)

        blk = d_r[...].reshape(2, EB // 128, 128)
        dp = blk[0]
        dn = blk[1]
        lp = jnp.log(jax.nn.sigmoid(dp) + EPS)
        ln = jnp.log(1.0 - jax.nn.sigmoid(dn) + EPS)
        accp[...] += jnp.sum(lp, axis=0, keepdims=True)
        accn[...] += jnp.sum(ln, axis=0, keepdims=True)

        @pl.when(i == negrid - 1)
        def _():
            pos_loss = -jnp.sum(accp[...]) / E
            neg_loss = -jnp.sum(accn[...]) / E
            recon = pos_loss + neg_loss
            scale = sc_r[0, 0]
            nkl = scale * jnp.sum(nkl_r[...])
            ckl = scale * jnp.sum(ckl_r[...])
            lane = jax.lax.broadcasted_iota(jnp.int32, (1, 128), 1)
            o_r[...] = jnp.where(
                lane == 0, recon,
                jnp.where(lane == 1, ckl, jnp.where(lane == 2, nkl, 0.0)))

    k_call = pl.pallas_call(
        body,
        grid=(negrid,),
        in_specs=[
            pl.BlockSpec((2, EB), lambda i: (0, i)),
            pl.BlockSpec((1, 128), lambda i: (0, 0)),
            pl.BlockSpec((1, 128), lambda i: (0, 0)),
            pl.BlockSpec((1, 1), lambda i: (0, 0), memory_space=pltpu.SMEM),
        ],
        out_specs=pl.BlockSpec((1, 128), lambda i: (0, 0)),
        out_shape=jax.ShapeDtypeStruct((1, 128), jnp.float32),
        scratch_shapes=[pltpu.VMEM((1, 128), jnp.float32),
                        pltpu.VMEM((1, 128), jnp.float32)],
    )
    scale = jnp.full((1, 1), -0.5e-07, jnp.float32) * jnp.float32(num_graphs)
    return k_call(d2, nkl_part, ckl_part, scale)


def kernel(x, edge_index, batch, num_graphs, W0, b0, W1, b1, Wnm, bnm, Wnl, bnl,
           Wcm, bcm, Wcl, bcl, Wd1, bd1, Wd2, bd2):
    src = edge_index[0]
    dst = edge_index[1]
    batch2 = batch.reshape(NBLK, 1, NB)
    kz = jax.random.key(42)
    eps_n = jax.random.normal(jax.random.fold_in(kz, 0), (N, 128), jnp.float32)
    eps_g = jax.random.normal(jax.random.fold_in(kz, 1), (NG, 128), jnp.float32)
    neg = jax.random.randint(jax.random.fold_in(kz, 2), (2, E), 0, N,
                             dtype=jnp.int32)

    p = _sc_agg(x, src, dst, 128, colsplit=False)
    h1 = _tc_layer(x, p[:N], p[NP:NP + N], W0, b0)

    p = _sc_agg(h1, src, dst, 64, colsplit=False)
    z_node, T, Smu, nkl_part = _tc_heads(
        h1, p[:N], p[NP:NP + N], W1, b1, Wnm, bnm, Wnl, bnl, Wcm, bcm,
        Wcl, bcl, eps_n, batch2)

    z_class, ckl_part = _tc_zclass(T, Smu, eps_g, batch2)

    tbl = jnp.concatenate([z_node, z_class], axis=0)
    srcs2 = jnp.concatenate([src, src + N])
    p = _sc_agg(tbl, srcs2, dst, 128, colsplit=True)

    ts, td = _tc_decoder(z_node, z_class, p[:N], p[NP:NP + N],
                         Wd1[:128], Wd1[128:], bd1, Wd2, Wd2.T, bd2)

    ne_pad = 655360
    zpad_i = jnp.zeros((ne_pad - 2 * E,), jnp.int32)
    ia = jnp.concatenate([src, neg[0], zpad_i])
    ib = jnp.concatenate([dst, neg[1], zpad_i])
    d = _sc_edge_dot(ts, td, ia, ib, ne_pad)

    out_row = _tc_final(d[:2 * E].reshape(2, E), nkl_part, ckl_part,
                        num_graphs)
    return out_row[0, :3]
